# baseline (speedup 1.0000x reference)
import functools

import jax
import jax.numpy as jnp
from jax import lax
from jax.experimental import pallas as pl
from jax.experimental.pallas import tpu as pltpu
from jax.experimental.pallas import tpu_sc as plsc

B = 32
K = 1024
DIM = 2048
KNN = 8
N_CLASS = 16
INV_T = 20.0

NC = 2
NS = 16
L = 16

CH = 16
CHD = CH * DIM
NBUF = 2
NCHUNK = K // CH
STEPS = NCHUNK // NBUF
DGRP = DIM // L


_mesh = plsc.VectorSubcoreMesh(core_axis_name="c", subcore_axis_name="s")


@functools.partial(
    pl.kernel,
    out_type=jax.ShapeDtypeStruct((B, N_CLASS), jnp.float32),
    mesh=_mesh,
    compiler_params=pltpu.CompilerParams(needs_layout_passes=False),
    scratch_types=[
        pltpu.VMEM((DIM,), jnp.float32),
        pltpu.VMEM((K,), jnp.int32),
        pltpu.VMEM((CHD,), jnp.float32),
        pltpu.VMEM((CHD,), jnp.float32),
        pltpu.VMEM((N_CLASS,), jnp.float32),
        pltpu.SemaphoreType.DMA,
        pltpu.SemaphoreType.DMA,
    ],
)
def _anchor_knn(logits_hbm, qa_hbm, ql_hbm, out_hbm,
                l_ref, lab_ref, buf0, buf1, outv, sem0, sem1):
    b = lax.axis_index("s") * NC + lax.axis_index("c")
    bufs = (buf0, buf1)
    sems = (sem0, sem1)

    pltpu.sync_copy(logits_hbm.at[b], l_ref)
    pltpu.sync_copy(ql_hbm.at[b], lab_ref)

    for i in range(NBUF):
        pltpu.async_copy(qa_hbm.at[b, pl.ds(i * CHD, CHD)], bufs[i], sems[i])

    lanes = lax.iota(jnp.int32, L)
    row_base = lanes * DIM

    def chunk_dists(buf):
        def dim_body(j, accs):
            accs = list(accs)
            base = j * L
            lvec = l_ref[pl.ds(base, L)]
            for t in range(L):
                g = plsc.load_gather(buf, [row_base + (base + t)])
                d = g - lvec[t]
                accs[t % 4] = accs[t % 4] + d * d
            return tuple(accs)

        z = jnp.zeros((L,), jnp.float32)
        a0, a1, a2, a3 = lax.fori_loop(0, DGRP, dim_body, (z, z, z, z))
        return (a0 + a1) + (a2 + a3)

    def step(s, carry):
        top_d, top_l = carry
        for i in range(NBUF):
            k = s * NBUF + i
            src = qa_hbm.at[b, pl.ds(k * CHD, CHD)]
            pltpu.make_async_copy(src, bufs[i], sems[i]).wait()

            dvec = chunk_dists(bufs[i])
            lab16 = lab_ref[pl.ds(k * CH, L)]

            nk = k + NBUF

            @pl.when(nk < NCHUNK)
            def _():
                pltpu.async_copy(
                    qa_hbm.at[b, pl.ds(nk * CHD, CHD)], bufs[i], sems[i])

            sd, sl = plsc.sort_key_val(dvec, lab16)
            sdr = jnp.flip(sd)
            slr = jnp.flip(sl)
            sel = top_d <= sdr
            md = jnp.where(sel, top_d, sdr)
            ml = jnp.where(sel, top_l, slr)
            top_d, top_l = plsc.sort_key_val(md, ml)
        return top_d, top_l

    top_d0 = jnp.full((L,), 3.0e38, jnp.float32)
    top_l0 = jnp.zeros((L,), jnp.int32)
    top_d, top_l = lax.fori_loop(0, STEPS, step, (top_d0, top_l0))

    valid = lanes < KNN
    s = jnp.where(valid, -INV_T * top_d, -1e30)
    m = jnp.max(s)
    e = jnp.exp(s - m)
    tot = jnp.sum(e)
    w = e / tot

    acc = jnp.zeros((N_CLASS,), jnp.float32)
    for i in range(KNN):
        acc = acc + jnp.where(lanes == top_l[i], w[i], 0.0)
    outv[...] = acc
    pltpu.sync_copy(outv, out_hbm.at[b])


def kernel(logits, queue_anchor, queue_label):
    qa_flat = queue_anchor.reshape(B, K * DIM)
    return _anchor_knn(logits, qa_flat, queue_label)


# speedup vs baseline: 8.3186x; 8.3186x over previous
import functools

import jax
import jax.numpy as jnp
from jax import lax
from jax.experimental import pallas as pl
from jax.experimental.pallas import tpu as pltpu
from jax.experimental.pallas import tpu_sc as plsc

B = 32
K = 1024
DIM = 2048
KNN = 8
N_CLASS = 16
INV_T = 20.0

NC = 2
NS = 16
L = 16

CH = 16
NBUF = 2
NCHUNK = K // CH
STEPS = NCHUNK // NBUF
DGRP = DIM // L


_mesh = plsc.VectorSubcoreMesh(core_axis_name="c", subcore_axis_name="s")


@functools.partial(
    pl.kernel,
    out_type=jax.ShapeDtypeStruct((B, N_CLASS), jnp.float32),
    mesh=_mesh,
    compiler_params=pltpu.CompilerParams(needs_layout_passes=False),
    scratch_types=[
        pltpu.VMEM((DIM,), jnp.float32),
        pltpu.VMEM((K,), jnp.int32),
        pltpu.VMEM((CH, DIM), jnp.float32),
        pltpu.VMEM((CH, DIM), jnp.float32),
        pltpu.VMEM((N_CLASS,), jnp.float32),
        pltpu.SemaphoreType.DMA,
        pltpu.SemaphoreType.DMA,
    ],
)
def _anchor_knn(logits_hbm, qa_hbm, ql_hbm, out_hbm,
                l_ref, lab_ref, buf0, buf1, outv, sem0, sem1):
    b = lax.axis_index("s") * NC + lax.axis_index("c")
    bufs = (buf0, buf1)
    sems = (sem0, sem1)

    pltpu.sync_copy(logits_hbm.at[b], l_ref)
    pltpu.sync_copy(ql_hbm.at[b], lab_ref)

    for i in range(NBUF):
        pltpu.async_copy(qa_hbm.at[b, pl.ds(i * CH, CH), :], bufs[i], sems[i])

    lanes = lax.iota(jnp.int32, L)

    def chunk_dists(buf):
        def dim_body(j, accs):
            base = j * L
            lvec = l_ref[pl.ds(base, L)]
            out = []
            for a in range(CH):
                d = buf[a, pl.ds(base, L)] - lvec
                out.append(accs[a] + d * d)
            return tuple(out)

        z = jnp.zeros((L,), jnp.float32)
        accs = lax.fori_loop(0, DGRP, dim_body, (z,) * CH)
        dvec = jnp.zeros((L,), jnp.float32)
        for a in range(CH):
            dvec = jnp.where(lanes == a, jnp.sum(accs[a]), dvec)
        return dvec

    def step(s, carry):
        top_d, top_l = carry
        for i in range(NBUF):
            k = s * NBUF + i
            src = qa_hbm.at[b, pl.ds(k * CH, CH), :]
            pltpu.make_async_copy(src, bufs[i], sems[i]).wait()

            dvec = chunk_dists(bufs[i])
            lab16 = lab_ref[pl.ds(k * CH, L)]

            nk = k + NBUF

            @pl.when(nk < NCHUNK)
            def _():
                pltpu.async_copy(
                    qa_hbm.at[b, pl.ds(nk * CH, CH), :], bufs[i], sems[i])

            sd, sl = plsc.sort_key_val(dvec, lab16)
            sdr = jnp.flip(sd)
            slr = jnp.flip(sl)
            sel = top_d <= sdr
            md = jnp.where(sel, top_d, sdr)
            ml = jnp.where(sel, top_l, slr)
            top_d, top_l = plsc.sort_key_val(md, ml)
        return top_d, top_l

    top_d0 = jnp.full((L,), 3.0e38, jnp.float32)
    top_l0 = jnp.zeros((L,), jnp.int32)
    top_d, top_l = lax.fori_loop(0, STEPS, step, (top_d0, top_l0))

    valid = lanes < KNN
    s = jnp.where(valid, -INV_T * top_d, -1e30)
    m = jnp.max(s)
    e = jnp.exp(s - m)
    tot = jnp.sum(e)
    w = e / tot

    acc = jnp.zeros((N_CLASS,), jnp.float32)
    for i in range(KNN):
        acc = acc + jnp.where(lanes == top_l[i], w[i], 0.0)
    outv[...] = acc
    pltpu.sync_copy(outv, out_hbm.at[b])


def kernel(logits, queue_anchor, queue_label):
    return _anchor_knn(logits, queue_anchor, queue_label)


# speedup vs baseline: 9.7887x; 1.1767x over previous
import functools

import jax
import jax.numpy as jnp
from jax import lax
from jax.experimental import pallas as pl
from jax.experimental.pallas import tpu as pltpu
from jax.experimental.pallas import tpu_sc as plsc

B = 32
K = 1024
DIM = 2048
KNN = 8
N_CLASS = 16
INV_T = 20.0

NC = 2
NS = 16
L = 16

CH = 16
NBUF = 3
NCHUNK = K // CH
STEPS = (NCHUNK - 1) // NBUF
UNROLL = 2
DGRP = DIM // (L * UNROLL)


_mesh = plsc.VectorSubcoreMesh(core_axis_name="c", subcore_axis_name="s")


@functools.partial(
    pl.kernel,
    out_type=jax.ShapeDtypeStruct((B, N_CLASS), jnp.float32),
    mesh=_mesh,
    compiler_params=pltpu.CompilerParams(needs_layout_passes=False),
    scratch_types=[
        pltpu.VMEM((DIM,), jnp.float32),
        pltpu.VMEM((K,), jnp.int32),
        pltpu.VMEM((CH, DIM), jnp.float32),
        pltpu.VMEM((CH, DIM), jnp.float32),
        pltpu.VMEM((CH, DIM), jnp.float32),
        pltpu.VMEM((N_CLASS,), jnp.float32),
        pltpu.SemaphoreType.DMA,
        pltpu.SemaphoreType.DMA,
        pltpu.SemaphoreType.DMA,
    ],
)
def _anchor_knn(logits_hbm, qa_hbm, ql_hbm, out_hbm,
                l_ref, lab_ref, buf0, buf1, buf2, outv, sem0, sem1, sem2):
    b = lax.axis_index("s") * NC + lax.axis_index("c")
    bufs = (buf0, buf1, buf2)
    sems = (sem0, sem1, sem2)

    pltpu.sync_copy(logits_hbm.at[b], l_ref)
    pltpu.sync_copy(ql_hbm.at[b], lab_ref)

    for i in range(NBUF):
        pltpu.async_copy(qa_hbm.at[b, pl.ds(i * CH, CH), :], bufs[i], sems[i])

    lanes = lax.iota(jnp.int32, L)

    def chunk_dists(buf):
        def dim_body(j, accs):
            accs = list(accs)
            for u in range(UNROLL):
                base = (j * UNROLL + u) * L
                lvec = l_ref[pl.ds(base, L)]
                for a in range(CH):
                    d = buf[a, pl.ds(base, L)] - lvec
                    accs[a] = accs[a] + d * d
            return tuple(accs)

        z = jnp.zeros((L,), jnp.float32)
        accs = lax.fori_loop(0, DGRP, dim_body, (z,) * CH)
        dvec = jnp.zeros((L,), jnp.float32)
        for a in range(CH):
            dvec = jnp.where(lanes == a, jnp.sum(accs[a]), dvec)
        return dvec

    def consume(k, i, top_d, top_l, refill):
        src = qa_hbm.at[b, pl.ds(k * CH, CH), :]
        pltpu.make_async_copy(src, bufs[i], sems[i]).wait()

        dvec = chunk_dists(bufs[i])
        lab16 = lab_ref[pl.ds(k * CH, L)]

        if refill:
            nk = k + NBUF

            @pl.when(nk < NCHUNK)
            def _():
                pltpu.async_copy(
                    qa_hbm.at[b, pl.ds(nk * CH, CH), :], bufs[i], sems[i])

        sd, sl = plsc.sort_key_val(dvec, lab16)
        sdr = jnp.flip(sd)
        slr = jnp.flip(sl)
        sel = top_d <= sdr
        md = jnp.where(sel, top_d, sdr)
        ml = jnp.where(sel, top_l, slr)
        return plsc.sort_key_val(md, ml)

    def step(s, carry):
        top_d, top_l = carry
        for i in range(NBUF):
            top_d, top_l = consume(s * NBUF + i, i, top_d, top_l, refill=True)
        return top_d, top_l

    top_d0 = jnp.full((L,), 3.0e38, jnp.float32)
    top_l0 = jnp.zeros((L,), jnp.int32)
    top_d, top_l = lax.fori_loop(0, STEPS, step, (top_d0, top_l0))
    top_d, top_l = consume(NCHUNK - 1, 0, top_d, top_l, refill=False)

    valid = lanes < KNN
    s = jnp.where(valid, -INV_T * top_d, -1e30)
    m = jnp.max(s)
    e = jnp.exp(s - m)
    tot = jnp.sum(e)
    w = e / tot

    acc = jnp.zeros((N_CLASS,), jnp.float32)
    for i in range(KNN):
        acc = acc + jnp.where(lanes == top_l[i], w[i], 0.0)
    outv[...] = acc
    pltpu.sync_copy(outv, out_hbm.at[b])


def kernel(logits, queue_anchor, queue_label):
    return _anchor_knn(logits, queue_anchor, queue_label)
